# baseline (speedup 1.0000x reference)
import functools

import jax
import jax.numpy as jnp
from jax import lax
from jax.experimental import pallas as pl
from jax.experimental.pallas import tpu as pltpu
from jax.experimental.pallas import tpu_sc as plsc

_EMBED = 128
_LANES = 16

_mesh = plsc.VectorSubcoreMesh(
    core_axis_name="c", subcore_axis_name="s", num_cores=1, num_subcores=1)


@functools.partial(
    pl.kernel,
    out_type=jax.ShapeDtypeStruct((_LANES,), jnp.float32),
    mesh=_mesh,
    compiler_params=pltpu.CompilerParams(needs_layout_passes=False),
    scratch_types=[
        pltpu.VMEM((1,), jnp.int32),
        pltpu.VMEM((1,), jnp.int32),
        pltpu.VMEM((1, _EMBED), jnp.float32),
        pltpu.VMEM((1, _EMBED), jnp.float32),
        pltpu.VMEM((_LANES,), jnp.float32),
        pltpu.SemaphoreType.DMA,
        pltpu.SemaphoreType.DMA,
    ],
)
def _sc_skipgram(ti_hbm, ci_hbm, w_hbm, c_hbm, out_hbm,
                 ti_v, ci_v, wrow_v, crow_v, out_v, sem_w, sem_c):
    cp_ti = pltpu.async_copy(ti_hbm, ti_v, sem_w)
    cp_ci = pltpu.async_copy(ci_hbm, ci_v, sem_c)
    cp_ti.wait()
    cp_w = pltpu.async_copy(w_hbm.at[ti_v], wrow_v, sem_w)
    cp_ci.wait()
    cp_c = pltpu.async_copy(c_hbm.at[ci_v], crow_v, sem_c)
    cp_w.wait()
    cp_c.wait()
    acc = jnp.zeros((_LANES,), jnp.float32)
    for j in range(_EMBED // _LANES):
        acc = acc + (wrow_v[0, pl.ds(j * _LANES, _LANES)]
                     * crow_v[0, pl.ds(j * _LANES, _LANES)])
    score = jnp.sum(acc)
    sv = jnp.full((_LANES,), score, jnp.float32)
    out_v[...] = 1.0 / (1.0 + jnp.exp(-sv))
    pltpu.sync_copy(out_v, out_hbm)


def kernel(tgt_word, ctx_word, w, c):
    ti = jnp.reshape(tgt_word.astype(jnp.int32), (1,))
    ci = jnp.reshape(ctx_word.astype(jnp.int32), (1,))
    out = _sc_skipgram(ti, ci, w, c)
    return out[0]


# speedup vs baseline: 1.0165x; 1.0165x over previous
import functools

import jax
import jax.numpy as jnp
from jax import lax
from jax.experimental import pallas as pl
from jax.experimental.pallas import tpu as pltpu
from jax.experimental.pallas import tpu_sc as plsc

_EMBED = 128
_LANES = 16

_mesh = plsc.VectorSubcoreMesh(
    core_axis_name="c", subcore_axis_name="s", num_cores=1, num_subcores=1)


@functools.partial(
    pl.kernel,
    out_type=jax.ShapeDtypeStruct((_LANES,), jnp.float32),
    mesh=_mesh,
    compiler_params=pltpu.CompilerParams(needs_layout_passes=False),
    scratch_types=[
        pltpu.VMEM((_LANES,), jnp.int32),
        pltpu.VMEM((1, _EMBED), jnp.float32),
        pltpu.VMEM((1, _EMBED), jnp.float32),
        pltpu.VMEM((_LANES,), jnp.float32),
        pltpu.SemaphoreType.DMA,
        pltpu.SemaphoreType.DMA,
    ],
)
def _sc_skipgram(idx_hbm, w_hbm, c_hbm, out_hbm,
                 idx_v, wrow_v, crow_v, out_v, sem_w, sem_c):
    cp_idx = pltpu.async_copy(idx_hbm, idx_v, sem_w)
    cp_idx.wait()
    cp_w = pltpu.async_copy(w_hbm.at[idx_v.at[pl.ds(0, 1)]], wrow_v, sem_w)
    cp_c = pltpu.async_copy(c_hbm.at[idx_v.at[pl.ds(8, 1)]], crow_v, sem_c)
    cp_w.wait()
    cp_c.wait()
    acc = jnp.zeros((_LANES,), jnp.float32)
    for j in range(_EMBED // _LANES):
        acc = acc + (wrow_v[0, pl.ds(j * _LANES, _LANES)]
                     * crow_v[0, pl.ds(j * _LANES, _LANES)])
    score = jnp.sum(acc)
    sv = jnp.full((_LANES,), score, jnp.float32)
    out_v[...] = 1.0 / (1.0 + jnp.exp(-sv))
    pltpu.sync_copy(out_v, out_hbm)


def kernel(tgt_word, ctx_word, w, c):
    idx = jnp.zeros((_LANES,), jnp.int32)
    idx = idx.at[0].set(tgt_word.astype(jnp.int32))
    idx = idx.at[8].set(ctx_word.astype(jnp.int32))
    out = _sc_skipgram(idx, w, c)
    return out[0]
